# Initial kernel scaffold; baseline (speedup 1.0000x reference)
import jax
import jax.numpy as jnp
from jax.experimental import pallas as pl


def kernel(h, edge_index, edge_mask, snorm_n, eps, W1, b1, W2, b2):
    raise NotImplementedError("write your pallas kernel here")


# speedup vs baseline: 4.6167x; 4.6167x over previous
import functools

import jax
import jax.numpy as jnp
from jax import lax
from jax.experimental import pallas as pl
from jax.experimental.pallas import tpu as pltpu
from jax.experimental.pallas import tpu_sc as plsc

D = 128
NC = 2
NS = 16
NW = NC * NS
K = 128


def _sc_segment_sum(h, src_r, dst_r, mask_r, ch):
    n = h.shape[0]
    rows_per_sub = -(-n // (NS * 8)) * 8
    np_ = rows_per_sub * NS
    full = (rows_per_sub // K) * K
    rem = rows_per_sub - full
    mesh = plsc.VectorSubcoreMesh(core_axis_name="c", subcore_axis_name="s")

    @functools.partial(
        pl.kernel,
        out_type=jax.ShapeDtypeStruct((NC, np_, D), jnp.float32),
        mesh=mesh,
        scratch_types=[
            pltpu.VMEM((ch, K), jnp.int32),
            pltpu.VMEM((ch, K), jnp.int32),
            pltpu.VMEM((ch * K,), jnp.float32),
            pltpu.VMEM((K, D), jnp.float32),
            pltpu.VMEM_SHARED((np_, D), jnp.float32),
            pltpu.SemaphoreType.DMA,
        ],
    )
    def seg(h_hbm, src_hbm, dst_hbm, mask_hbm, out_hbm,
            src_v, dst_v, mask_v, rows_v, acc_s, sem):
        cid = lax.axis_index("c")
        sid = lax.axis_index("s")
        wid = cid * NS + sid

        pltpu.sync_copy(src_hbm.at[wid], src_v)
        pltpu.sync_copy(dst_hbm.at[wid], dst_v)
        pltpu.sync_copy(mask_hbm.at[wid], mask_v)

        def zrow(i, _):
            for j in range(D // 16):
                rows_v[i, pl.ds(j * 16, 16)] = jnp.zeros((16,), jnp.float32)
            return 0
        lax.fori_loop(0, K, zrow, 0)
        base = sid * rows_per_sub
        for t in range(full // K):
            pltpu.sync_copy(rows_v, acc_s.at[pl.ds(base + t * K, K)])
        if rem:
            pltpu.sync_copy(rows_v.at[pl.ds(0, rem)],
                            acc_s.at[pl.ds(base + full, rem)])
        plsc.subcore_barrier()

        def chunk_body(c, _):
            pltpu.async_copy(h_hbm.at[src_v.at[c]], rows_v, sem).wait()

            def escale(g, _):
                mvec = mask_v[pl.ds((c * (K // 16) + g) * 16, 16)]
                for l in range(16):
                    m = jnp.full((16,), mvec[l])
                    e = g * 16 + l
                    for j in range(D // 16):
                        sl = pl.ds(j * 16, 16)
                        rows_v[e, sl] = rows_v[e, sl] * m
                return 0
            lax.fori_loop(0, K // 16, escale, 0)

            pltpu.sync_copy(rows_v, acc_s.at[dst_v.at[c]], add=True)
            return 0
        lax.fori_loop(0, ch, chunk_body, 0)
        plsc.subcore_barrier()

        pltpu.sync_copy(acc_s.at[pl.ds(base, rows_per_sub)],
                        out_hbm.at[cid].at[pl.ds(base, rows_per_sub)])

    return seg(h, src_r, dst_r, mask_r)


def _tc_finish(h, parts, eps, W1, b1, W2, b2):
    n = h.shape[0]
    bn = 1000

    def body(eps_ref, h_ref, p_ref, w1_ref, b1_ref, w2_ref, b2_ref, o_ref):
        x = (1.0 + eps_ref[0]) * h_ref[...] + p_ref[0] + p_ref[1]
        y = jnp.dot(x, w1_ref[...], preferred_element_type=jnp.float32)
        y = jnp.maximum(y + b1_ref[...], 0.0)
        z = jnp.dot(y, w2_ref[...], preferred_element_type=jnp.float32)
        o_ref[...] = jnp.maximum(z + b2_ref[...], 0.0)

    return pl.pallas_call(
        body,
        grid=(n // bn,),
        in_specs=[
            pl.BlockSpec(memory_space=pltpu.SMEM),
            pl.BlockSpec((bn, D), lambda i: (i, 0)),
            pl.BlockSpec((NC, bn, D), lambda i: (0, i, 0)),
            pl.BlockSpec((D, D), lambda i: (0, 0)),
            pl.BlockSpec((1, D), lambda i: (0, 0)),
            pl.BlockSpec((D, D), lambda i: (0, 0)),
            pl.BlockSpec((1, D), lambda i: (0, 0)),
        ],
        out_specs=pl.BlockSpec((bn, D), lambda i: (i, 0)),
        out_shape=jax.ShapeDtypeStruct((n, D), jnp.float32),
    )(eps, h, parts, W1, b1.reshape(1, D), W2, b2.reshape(1, D))


def kernel(h, edge_index, edge_mask, snorm_n, eps, W1, b1, W2, b2):
    e = edge_index.shape[1]
    ch = -(-e // (NW * K))
    pad = NW * ch * K - e
    src = jnp.pad(edge_index[0], (0, pad)).reshape(NW, ch, K)
    dst = jnp.pad(edge_index[1], (0, pad)).reshape(NW, ch, K)
    mask = jnp.pad(edge_mask, (0, pad)).reshape(NW, ch * K)
    parts = _sc_segment_sum(h, src, dst, mask, ch)
    return _tc_finish(h, parts, eps, W1, b1, W2, b2)
